# baseline (speedup 1.0000x reference)
import functools
import math

import jax
import jax.numpy as jnp
import numpy as np
from jax import lax
from jax.experimental import pallas as pl
from jax.experimental.pallas import tpu as pltpu
from jax.experimental.pallas import tpu_sc as plsc

N_NODES = 128
B = 256
N = B * N_NODES
E = 524288
D = 128

NC, NS = 2, 16
SUB = 2048
NSUB_E = E // NS // SUB
NSUB = NSUB_E + 1
CHUNK_ROWS = N // NC
CHUNK = CHUNK_ROWS * (D // 2)
ZB = 8192


def _hist_body(edge_hbm, nid_hbm, c_hbm,
               nid_v, src_v, dst_v, idx_v, val_v, zero_v, shared,
               esem, ssem):
    c = lax.axis_index("c")
    s = lax.axis_index("s")
    tile_base = s * (E // NS)
    slice0 = s * (CHUNK // NS)

    def edge_fetch(b):
        ib = tile_base + b * SUB
        return (pltpu.async_copy(edge_hbm.at[0, pl.ds(ib, SUB)],
                                 src_v.at[b % 2], esem),
                pltpu.async_copy(edge_hbm.at[1, pl.ds(ib, SUB)],
                                 dst_v.at[b % 2], esem))

    edesc = edge_fetch(0)
    pltpu.sync_copy(nid_hbm, nid_v)

    z16 = jnp.zeros((16,), jnp.int32)

    def zb_body(i, _):
        zero_v[pl.ds(i * 16, 16)] = z16
        return 0

    lax.fori_loop(0, ZB // 16, zb_body, 0)

    zdescs = [pltpu.async_copy(
        zero_v, shared.at[pl.ds(slice0 + r * ZB, ZB)], esem)
        for r in range(CHUNK // NS // ZB)]
    for d in zdescs:
        d.wait()
    plsc.subcore_barrier()

    sdesc = [[], []]

    def fire_scatters(p):
        for r in range(SUB // 128):
            sdesc[p].append(pltpu.async_copy(
                val_v.at[p, r],
                shared.at[plsc.Indices(idx_v.at[p, r], ignored_value=-1)],
                ssem, add=True))

    def drain_scatters(p):
        for d in sdesc[p]:
            d.wait()
        sdesc[p] = []

    base = c * CHUNK

    def unpack_nid(i16):
        return plsc.load_gather(nid_v, [i16])

    def emit(p, g, d16, n16):
        widx = lax.bitwise_or(lax.shift_left(d16, 6),
                              lax.bitwise_and(n16, 63))
        rel = widx - base
        inr = rel.astype(jnp.uint32) < CHUNK
        hi4 = lax.shift_left(lax.shift_right_logical(n16, 6), 4)
        row = g // 8
        col = pl.ds((g % 8) * 16, 16)
        idx_v[p, row, col] = jnp.where(inr, rel, -1)
        val_v[p, row, col] = lax.shift_left(1, hi4)

    for b in range(NSUB_E):
        for d in edesc:
            d.wait()
        if b + 1 < NSUB_E:
            edesc = edge_fetch(b + 1)
        p = b % 2
        drain_scatters(p)

        @plsc.parallel_loop(0, SUB // 16, 1, unroll=4)
        def grp_body(g):
            s16 = src_v[p, pl.ds(g * 16, 16)]
            d16 = dst_v[p, pl.ds(g * 16, 16)]
            emit(p, g, d16, unpack_nid(s16))

        fire_scatters(p)

    p = NSUB_E % 2
    drain_scatters(p)
    self_base = s * (N // NS)
    lane = lax.iota(jnp.int32, 16)

    @plsc.parallel_loop(0, SUB // 16, 1, unroll=4)
    def self_body(g):
        i16 = lane + (self_base + g * 16)
        emit(p, g, i16, unpack_nid(i16))

    fire_scatters(p)

    drain_scatters(0)
    drain_scatters(1)
    plsc.subcore_barrier()
    pltpu.sync_copy(shared.at[pl.ds(slice0, CHUNK // NS)],
                    c_hbm.at[pl.ds(base + slice0, CHUNK // NS)])


def _build_counts(edge_index, node_ids):
    mesh = plsc.VectorSubcoreMesh(core_axis_name="c", subcore_axis_name="s")
    return pl.kernel(
        _hist_body,
        out_type=jax.ShapeDtypeStruct((N * (D // 2),), jnp.int32),
        mesh=mesh,
        compiler_params=pltpu.CompilerParams(needs_layout_passes=False),
        scratch_types=[
            pltpu.VMEM((N,), jnp.int32),
            pltpu.VMEM((2, SUB), jnp.int32),
            pltpu.VMEM((2, SUB), jnp.int32),
            pltpu.VMEM((2, SUB // 128, 128), jnp.int32),
            pltpu.VMEM((2, SUB // 128, 128), jnp.int32),
            pltpu.VMEM((ZB,), jnp.int32),
            pltpu.VMEM_SHARED((CHUNK,), jnp.int32),
            pltpu.SemaphoreType.DMA,
            pltpu.SemaphoreType.DMA,
        ],
    )(edge_index, node_ids)


TRIU = N_NODES * (N_NODES - 1) // 2
GPT = B // (NC * NS)


def _pack_body(nb, dp_hbm, idx_hbm, out_hbm, idx_v, dp_a, dp_b, out_a, out_b,
               dsem, osem):
    c = lax.axis_index("c")
    s = lax.axis_index("s")
    gpt = nb // (NC * NS)
    g0 = c * (nb // NC) + s * gpt
    pltpu.sync_copy(idx_hbm, idx_v)
    nn2 = N_NODES * N_NODES
    dp_v = [dp_a, dp_b]
    out_v = [out_a, out_b]
    ddesc = [None, None]
    odesc = [None, None]
    ddesc[0] = pltpu.async_copy(dp_hbm.at[pl.ds(g0 * nn2, nn2)], dp_a, dsem)
    for k in range(gpt):
        p = k % 2
        ddesc[p].wait()
        if k + 1 < gpt:
            ddesc[1 - p] = pltpu.async_copy(
                dp_hbm.at[pl.ds((g0 + k + 1) * nn2, nn2)], dp_v[1 - p], dsem)
        if odesc[p] is not None:
            odesc[p].wait()
        dpb = dp_v[p]
        outb = out_v[p]

        @plsc.parallel_loop(0, TRIU // 16, 1, unroll=4)
        def gat(q):
            i16 = idx_v[pl.ds(q * 16, 16)]
            outb[0, pl.ds(q * 16, 16)] = plsc.load_gather(dpb, [i16])

        odesc[p] = pltpu.async_copy(outb,
                                    out_hbm.at[pl.ds(g0 + k, 1), :], osem)
    for d in odesc:
        d.wait()


def _pack_triu(dp_flat, flat_idx, nb):
    mesh = plsc.VectorSubcoreMesh(core_axis_name="c", subcore_axis_name="s")
    return pl.kernel(
        functools.partial(_pack_body, nb),
        out_type=jax.ShapeDtypeStruct((nb, TRIU), jnp.float32),
        mesh=mesh,
        compiler_params=pltpu.CompilerParams(needs_layout_passes=False),
        scratch_types=[
            pltpu.VMEM((TRIU,), jnp.int32),
            pltpu.VMEM((N_NODES * N_NODES,), jnp.float32),
            pltpu.VMEM((N_NODES * N_NODES,), jnp.float32),
            pltpu.VMEM((1, TRIU), jnp.float32),
            pltpu.VMEM((1, TRIU), jnp.float32),
            pltpu.SemaphoreType.DMA,
            pltpu.SemaphoreType.DMA,
        ],
    )(dp_flat, flat_idx)


BLK = 1024
G_PER_BLK = BLK // N_NODES
GRID = N // BLK
GRID_H = GRID // 2
BH = B // 2


def _ln(h, g, b):
    m = jnp.mean(h, axis=-1, keepdims=True)
    v = jnp.mean((h - m) ** 2, axis=-1, keepdims=True)
    return (h - m) * lax.rsqrt(v + 1e-5) * g + b


def _mm(a, b):
    return jnp.dot(a, b, preferred_element_type=jnp.float32)


def _dense_body(c_ref, osums_ref, emb_ref, w1_ref, b1_ref, lng_ref, lnb_ref,
                w2_ref, b2_ref, sw1_ref, sb1_ref, sw2_ref, sb2_ref,
                ng_ref, nb_ref, ew1_ref, eb1_ref, elng_ref, elnb_ref,
                ew2_ref, eb2_ref, dp_ref, sums_ref, exit_ref, m_s, sums_s):
    i = pl.program_id(0)

    @pl.when(i == 0)
    def _():
        m_s[:] = jnp.dot(emb_ref[:], w1_ref[:],
                         preferred_element_type=jnp.float32)

    w = c_ref[:]
    lo = lax.bitwise_and(w, 0xFFFF).astype(jnp.float32)
    hi = lax.shift_right_logical(w, 16).astype(jnp.float32)
    cnt = jnp.concatenate(
        [jnp.concatenate([lo[:, :64], hi[:, :64]], axis=1),
         jnp.concatenate([lo[:, 64:], hi[:, 64:]], axis=1)], axis=0)
    h = _mm(cnt, m_s[:]) + b1_ref[:]
    h = _ln(h, lng_ref[:], lnb_ref[:])
    h = jnp.maximum(h, 0.0)
    h = _mm(h, w2_ref[:]) + b2_ref[:]
    h = _mm(h, sw1_ref[:]) + sb1_ref[:]
    h = jnp.maximum(h, 0.0)
    h = _mm(h, sw2_ref[:]) + sb2_ref[:]
    x = _ln(h, ng_ref[:], nb_ref[:])

    scale = 1.0 / math.sqrt(D)
    half = BLK // 2
    hn = N_NODES // 2
    for g in range(G_PER_BLK):
        xg = jnp.concatenate([x[g * hn:(g + 1) * hn, :],
                              x[half + g * hn:half + (g + 1) * hn, :]],
                             axis=0)
        sg = jnp.sum(xg, axis=0)
        sums_s[i * G_PER_BLK + g, :] = sg
        sums_ref[g, :] = sg
        dp_ref[g, :, :] = lax.dot_general(
            xg, xg, (((1,), (1,)), ((), ())),
            preferred_element_type=jnp.float32) * scale

    @pl.when(i == GRID_H - 1)
    def _():
        means = jnp.concatenate([osums_ref[:], sums_s[:]],
                                axis=0) * (1.0 / N_NODES)
        e = jnp.dot(means, ew1_ref[:],
                    preferred_element_type=jnp.float32) + eb1_ref[:]
        e = _ln(e, elng_ref[:], elnb_ref[:])
        e = jnp.maximum(e, 0.0)
        exit_ref[:] = jnp.dot(e, ew2_ref[:],
                              preferred_element_type=jnp.float32) + eb2_ref[:]


def _dense_stage(counts, half, osums, emb, w1, b1, lng, lnb, w2, b2,
                 sw1, sb1, sw2, sb2, ng, nb,
                 ew1, eb1, elng, elnb, ew2, eb2):
    wspec = pl.BlockSpec((D, D), lambda i: (0, 0))
    bspec = pl.BlockSpec((1, D), lambda i: (0, 0))
    return pl.pallas_call(
        _dense_body,
        grid=(GRID_H,),
        in_specs=[
            pl.BlockSpec((BLK // 2, D), lambda i: (i + half * GRID_H, 0)),
            pl.BlockSpec((BH, D), lambda i: (0, 0)),
            wspec, wspec, bspec, bspec, bspec,
            wspec, bspec, wspec, bspec, wspec, bspec,
            bspec, bspec,
            wspec, bspec, bspec, bspec,
            pl.BlockSpec((D, 1), lambda i: (0, 0)),
            pl.BlockSpec((1, 1), lambda i: (0, 0)),
        ],
        out_specs=[
            pl.BlockSpec((G_PER_BLK, N_NODES, N_NODES), lambda i: (i, 0, 0)),
            pl.BlockSpec((G_PER_BLK, D), lambda i: (i, 0)),
            pl.BlockSpec((B, 1), lambda i: (0, 0)),
        ],
        out_shape=[
            jax.ShapeDtypeStruct((BH, N_NODES, N_NODES), jnp.float32),
            jax.ShapeDtypeStruct((BH, D), jnp.float32),
            jax.ShapeDtypeStruct((B, 1), jnp.float32),
        ],
        scratch_shapes=[
            pltpu.VMEM((D, D), jnp.float32),
            pltpu.VMEM((BH, D), jnp.float32),
        ],
    )(counts, osums, emb, w1, b1, lng, lnb, w2, b2, sw1, sb1, sw2, sb2,
      ng, nb, ew1, eb1, elng, elnb, ew2, eb2)


def kernel(node_ids, edge_index, ptr, emb, gin_w1, gin_b1, gin_lng, gin_lnb,
           gin_w2, gin_b2, seq_w1, seq_b1, seq_w2, seq_b2, norm_g, norm_b,
           ex_w1, ex_b1, ex_lng, ex_lnb, ex_w2, ex_b2):
    del ptr
    node_ids = node_ids.astype(jnp.int32)

    counts = _build_counts(edge_index.astype(jnp.int32),
                           node_ids).reshape(N // 2, D)

    r2 = lambda v: v.reshape(1, D)
    wargs = (emb, gin_w1, r2(gin_b1), r2(gin_lng), r2(gin_lnb),
             gin_w2, r2(gin_b2), seq_w1, r2(seq_b1), seq_w2, r2(seq_b2),
             r2(norm_g), r2(norm_b),
             ex_w1, r2(ex_b1), r2(ex_lng), r2(ex_lnb), ex_w2,
             ex_b2.reshape(1, 1))
    zsums = jnp.zeros((BH, D), jnp.float32)
    dp1, sums1, _ = _dense_stage(counts, 0, zsums, *wargs)
    dp2, _, exit_action = _dense_stage(counts, 1, sums1, *wargs)

    i0, i1 = np.triu_indices(N_NODES, k=1)
    p0 = (i0 // 2) + (N_NODES // 2) * (i0 % 2)
    p1 = (i1 // 2) + (N_NODES // 2) * (i1 % 2)
    flat_idx = jnp.asarray(p0 * N_NODES + p1, dtype=jnp.int32)
    ea1 = _pack_triu(dp1.reshape(-1), flat_idx, BH)
    ea2 = _pack_triu(dp2.reshape(-1), flat_idx, BH)
    edge_actions = jnp.concatenate([ea1, ea2], axis=0)
    return jnp.concatenate([edge_actions, exit_action], axis=-1)


# speedup vs baseline: 1.0824x; 1.0824x over previous
import functools
import math

import jax
import jax.numpy as jnp
import numpy as np
from jax import lax
from jax.experimental import pallas as pl
from jax.experimental.pallas import tpu as pltpu
from jax.experimental.pallas import tpu_sc as plsc

N_NODES = 128
B = 256
N = B * N_NODES
E = 524288
D = 128

NC, NS = 2, 16
SUB = 2048
NSUB_E = E // NS // SUB
NSUB = NSUB_E + 1
CHUNK_ROWS = N // NC
CHUNK = CHUNK_ROWS * (D // 2)
ZB = 8192


def _hist_body(edge_hbm, nid_hbm, c_hbm,
               nid_v, src_v, dst_v, idx_v, val_v, zero_v, shared,
               esem, ssem):
    c = lax.axis_index("c")
    s = lax.axis_index("s")
    tile_base = s * (E // NS)
    slice0 = s * (CHUNK // NS)

    def edge_fetch(b):
        ib = tile_base + b * SUB
        return (pltpu.async_copy(edge_hbm.at[0, pl.ds(ib, SUB)],
                                 src_v.at[b % 2], esem),
                pltpu.async_copy(edge_hbm.at[1, pl.ds(ib, SUB)],
                                 dst_v.at[b % 2], esem))

    edesc = edge_fetch(0)
    pltpu.sync_copy(nid_hbm, nid_v)

    z16 = jnp.zeros((16,), jnp.int32)

    def zb_body(i, _):
        zero_v[pl.ds(i * 16, 16)] = z16
        return 0

    lax.fori_loop(0, ZB // 16, zb_body, 0)

    zdescs = [pltpu.async_copy(
        zero_v, shared.at[pl.ds(slice0 + r * ZB, ZB)], esem)
        for r in range(CHUNK // NS // ZB)]
    for d in zdescs:
        d.wait()
    plsc.subcore_barrier()

    sdesc = [[], []]

    def fire_scatters(p):
        for r in range(SUB // 128):
            sdesc[p].append(pltpu.async_copy(
                val_v.at[p, r],
                shared.at[plsc.Indices(idx_v.at[p, r], ignored_value=-1)],
                ssem, add=True))

    def drain_scatters(p):
        for d in sdesc[p]:
            d.wait()
        sdesc[p] = []

    base = c * CHUNK

    def unpack_nid(i16):
        return plsc.load_gather(nid_v, [i16])

    def emit(p, g, d16, n16):
        widx = lax.bitwise_or(lax.shift_left(d16, 6),
                              lax.bitwise_and(n16, 63))
        rel = widx - base
        inr = rel.astype(jnp.uint32) < CHUNK
        hi4 = lax.shift_left(lax.shift_right_logical(n16, 6), 4)
        row = g // 8
        col = pl.ds((g % 8) * 16, 16)
        idx_v[p, row, col] = jnp.where(inr, rel, -1)
        val_v[p, row, col] = lax.shift_left(1, hi4)

    for b in range(NSUB_E):
        for d in edesc:
            d.wait()
        if b + 1 < NSUB_E:
            edesc = edge_fetch(b + 1)
        p = b % 2
        drain_scatters(p)

        @plsc.parallel_loop(0, SUB // 16, 1, unroll=4)
        def grp_body(g):
            s16 = src_v[p, pl.ds(g * 16, 16)]
            d16 = dst_v[p, pl.ds(g * 16, 16)]
            emit(p, g, d16, unpack_nid(s16))

        fire_scatters(p)

    p = NSUB_E % 2
    drain_scatters(p)
    self_base = s * (N // NS)
    lane = lax.iota(jnp.int32, 16)

    @plsc.parallel_loop(0, SUB // 16, 1, unroll=4)
    def self_body(g):
        i16 = lane + (self_base + g * 16)
        emit(p, g, i16, unpack_nid(i16))

    fire_scatters(p)

    drain_scatters(0)
    drain_scatters(1)
    plsc.subcore_barrier()
    pltpu.sync_copy(shared.at[pl.ds(slice0, CHUNK // NS)],
                    c_hbm.at[pl.ds(base + slice0, CHUNK // NS)])


def _build_counts(edge_index, node_ids):
    mesh = plsc.VectorSubcoreMesh(core_axis_name="c", subcore_axis_name="s")
    return pl.kernel(
        _hist_body,
        out_type=jax.ShapeDtypeStruct((N * (D // 2),), jnp.int32),
        mesh=mesh,
        compiler_params=pltpu.CompilerParams(needs_layout_passes=False),
        scratch_types=[
            pltpu.VMEM((N,), jnp.int32),
            pltpu.VMEM((2, SUB), jnp.int32),
            pltpu.VMEM((2, SUB), jnp.int32),
            pltpu.VMEM((2, SUB // 128, 128), jnp.int32),
            pltpu.VMEM((2, SUB // 128, 128), jnp.int32),
            pltpu.VMEM((ZB,), jnp.int32),
            pltpu.VMEM_SHARED((CHUNK,), jnp.int32),
            pltpu.SemaphoreType.DMA,
            pltpu.SemaphoreType.DMA,
        ],
    )(edge_index, node_ids)


TRIU = N_NODES * (N_NODES - 1) // 2
GPT = B // (NC * NS)


def _pack_body(dp_hbm, idx_hbm, out_hbm, idx_v, dp_a, dp_b, out_a, out_b,
               dsem, osem):
    c = lax.axis_index("c")
    s = lax.axis_index("s")
    g0 = c * (B // NC) + s * GPT
    pltpu.sync_copy(idx_hbm, idx_v)
    nn2 = N_NODES * N_NODES
    dp_v = [dp_a, dp_b]
    out_v = [out_a, out_b]
    ddesc = [None, None]
    odesc = [None, None]
    ddesc[0] = pltpu.async_copy(dp_hbm.at[pl.ds(g0 * nn2, nn2)], dp_a, dsem)
    for k in range(GPT):
        p = k % 2
        ddesc[p].wait()
        if k + 1 < GPT:
            ddesc[1 - p] = pltpu.async_copy(
                dp_hbm.at[pl.ds((g0 + k + 1) * nn2, nn2)], dp_v[1 - p], dsem)
        if odesc[p] is not None:
            odesc[p].wait()
        dpb = dp_v[p]
        outb = out_v[p]

        @plsc.parallel_loop(0, TRIU // 16, 1, unroll=4)
        def gat(q):
            i16 = idx_v[pl.ds(q * 16, 16)]
            outb[0, pl.ds(q * 16, 16)] = plsc.load_gather(dpb, [i16])

        odesc[p] = pltpu.async_copy(outb,
                                    out_hbm.at[pl.ds(g0 + k, 1), :], osem)
    for d in odesc:
        d.wait()


def _pack_triu(dp_flat, flat_idx):
    mesh = plsc.VectorSubcoreMesh(core_axis_name="c", subcore_axis_name="s")
    return pl.kernel(
        _pack_body,
        out_type=jax.ShapeDtypeStruct((B, TRIU), jnp.float32),
        mesh=mesh,
        compiler_params=pltpu.CompilerParams(needs_layout_passes=False),
        scratch_types=[
            pltpu.VMEM((TRIU,), jnp.int32),
            pltpu.VMEM((N_NODES * N_NODES,), jnp.float32),
            pltpu.VMEM((N_NODES * N_NODES,), jnp.float32),
            pltpu.VMEM((1, TRIU), jnp.float32),
            pltpu.VMEM((1, TRIU), jnp.float32),
            pltpu.SemaphoreType.DMA,
            pltpu.SemaphoreType.DMA,
        ],
    )(dp_flat, flat_idx)


BLK = 1024
G_PER_BLK = BLK // N_NODES
GRID = N // BLK


def _ln(h, g, b):
    m = jnp.mean(h, axis=-1, keepdims=True)
    v = jnp.mean((h - m) ** 2, axis=-1, keepdims=True)
    return (h - m) * lax.rsqrt(v + 1e-5) * g + b


def _mm(a, b):
    return jnp.dot(a, b, preferred_element_type=jnp.float32)


def _dense_body(c_ref, emb_ref, w1_ref, b1_ref, lng_ref, lnb_ref,
                w2_ref, b2_ref, sw1_ref, sb1_ref, sw2_ref, sb2_ref,
                ng_ref, nb_ref, ew1_ref, eb1_ref, elng_ref, elnb_ref,
                ew2_ref, eb2_ref, dp_ref, exit_ref, m_s, sums_s):
    i = pl.program_id(0)

    @pl.when(i == 0)
    def _():
        m_s[:] = jnp.dot(emb_ref[:], w1_ref[:],
                         preferred_element_type=jnp.float32)

    w = c_ref[:]
    lo = lax.bitwise_and(w, 0xFFFF).astype(jnp.float32)
    hi = lax.shift_right_logical(w, 16).astype(jnp.float32)
    cnt = jnp.concatenate(
        [jnp.concatenate([lo[:, :64], hi[:, :64]], axis=1),
         jnp.concatenate([lo[:, 64:], hi[:, 64:]], axis=1)], axis=0)
    h = _mm(cnt, m_s[:]) + b1_ref[:]
    h = _ln(h, lng_ref[:], lnb_ref[:])
    h = jnp.maximum(h, 0.0)
    h = _mm(h, w2_ref[:]) + b2_ref[:]
    h = _mm(h, sw1_ref[:]) + sb1_ref[:]
    h = jnp.maximum(h, 0.0)
    h = _mm(h, sw2_ref[:]) + sb2_ref[:]
    x = _ln(h, ng_ref[:], nb_ref[:])

    scale = 1.0 / math.sqrt(D)
    half = BLK // 2
    hn = N_NODES // 2
    for g in range(G_PER_BLK):
        xg = jnp.concatenate([x[g * hn:(g + 1) * hn, :],
                              x[half + g * hn:half + (g + 1) * hn, :]],
                             axis=0)
        sums_s[i * G_PER_BLK + g, :] = jnp.sum(xg, axis=0)
        dp_ref[g, :, :] = lax.dot_general(
            xg, xg, (((1,), (1,)), ((), ())),
            preferred_element_type=jnp.float32) * scale

    @pl.when(i == GRID - 1)
    def _():
        means = sums_s[:] * (1.0 / N_NODES)
        e = jnp.dot(means, ew1_ref[:],
                    preferred_element_type=jnp.float32) + eb1_ref[:]
        e = _ln(e, elng_ref[:], elnb_ref[:])
        e = jnp.maximum(e, 0.0)
        exit_ref[:] = jnp.dot(e, ew2_ref[:],
                              preferred_element_type=jnp.float32) + eb2_ref[:]


def _dense_stage(counts, emb, w1, b1, lng, lnb, w2, b2,
                 sw1, sb1, sw2, sb2, ng, nb,
                 ew1, eb1, elng, elnb, ew2, eb2):
    wspec = pl.BlockSpec((D, D), lambda i: (0, 0))
    bspec = pl.BlockSpec((1, D), lambda i: (0, 0))
    return pl.pallas_call(
        _dense_body,
        grid=(GRID,),
        in_specs=[
            pl.BlockSpec((BLK // 2, D), lambda i: (i, 0)),
            wspec, wspec, bspec, bspec, bspec,
            wspec, bspec, wspec, bspec, wspec, bspec,
            bspec, bspec,
            wspec, bspec, bspec, bspec,
            pl.BlockSpec((D, 1), lambda i: (0, 0)),
            pl.BlockSpec((1, 1), lambda i: (0, 0)),
        ],
        out_specs=[
            pl.BlockSpec((G_PER_BLK, N_NODES, N_NODES), lambda i: (i, 0, 0)),
            pl.BlockSpec((B, 1), lambda i: (0, 0)),
        ],
        out_shape=[
            jax.ShapeDtypeStruct((B, N_NODES, N_NODES), jnp.float32),
            jax.ShapeDtypeStruct((B, 1), jnp.float32),
        ],
        scratch_shapes=[
            pltpu.VMEM((D, D), jnp.float32),
            pltpu.VMEM((B, D), jnp.float32),
        ],
    )(counts, emb, w1, b1, lng, lnb, w2, b2, sw1, sb1, sw2, sb2, ng, nb,
      ew1, eb1, elng, elnb, ew2, eb2)


def kernel(node_ids, edge_index, ptr, emb, gin_w1, gin_b1, gin_lng, gin_lnb,
           gin_w2, gin_b2, seq_w1, seq_b1, seq_w2, seq_b2, norm_g, norm_b,
           ex_w1, ex_b1, ex_lng, ex_lnb, ex_w2, ex_b2):
    del ptr
    node_ids = node_ids.astype(jnp.int32)

    counts = _build_counts(edge_index.astype(jnp.int32),
                           node_ids).reshape(N // 2, D)

    r2 = lambda v: v.reshape(1, D)
    dp, exit_action = _dense_stage(
        counts, emb, gin_w1, r2(gin_b1), r2(gin_lng), r2(gin_lnb),
        gin_w2, r2(gin_b2), seq_w1, r2(seq_b1), seq_w2, r2(seq_b2),
        r2(norm_g), r2(norm_b),
        ex_w1, r2(ex_b1), r2(ex_lng), r2(ex_lnb), ex_w2,
        ex_b2.reshape(1, 1))

    i0, i1 = np.triu_indices(N_NODES, k=1)
    p0 = (i0 // 2) + (N_NODES // 2) * (i0 % 2)
    p1 = (i1 // 2) + (N_NODES // 2) * (i1 % 2)
    flat_idx = jnp.asarray(p0 * N_NODES + p1, dtype=jnp.int32)
    edge_actions = _pack_triu(dp.reshape(-1), flat_idx)
    return jnp.concatenate([edge_actions, exit_action], axis=-1)


# speedup vs baseline: 1.2272x; 1.1338x over previous
import functools
import math

import jax
import jax.numpy as jnp
import numpy as np
from jax import lax
from jax.experimental import pallas as pl
from jax.experimental.pallas import tpu as pltpu
from jax.experimental.pallas import tpu_sc as plsc

N_NODES = 128
B = 256
N = B * N_NODES
E = 524288
D = 128

NC, NS = 2, 16
SUB = 2048
NSUB_E = E // NS // SUB
NSUB = NSUB_E + 1
CHUNK_ROWS = N // NC
CHUNK = CHUNK_ROWS * (D // 2)
ZB = 8192


def _hist_body(edge_hbm, nid_hbm, c_hbm,
               nid_v, src_v, dst_v, idx_v, val_v, zero_v, shared,
               esem, ssem):
    c = lax.axis_index("c")
    s = lax.axis_index("s")
    tile_base = s * (E // NS)
    slice0 = s * (CHUNK // NS)

    def edge_fetch(b):
        ib = tile_base + b * SUB
        return (pltpu.async_copy(edge_hbm.at[0, pl.ds(ib, SUB)],
                                 src_v.at[b % 2], esem),
                pltpu.async_copy(edge_hbm.at[1, pl.ds(ib, SUB)],
                                 dst_v.at[b % 2], esem))

    edesc = edge_fetch(0)
    pltpu.sync_copy(nid_hbm, nid_v)

    z16 = jnp.zeros((16,), jnp.int32)

    def zb_body(i, _):
        zero_v[pl.ds(i * 16, 16)] = z16
        return 0

    lax.fori_loop(0, ZB // 16, zb_body, 0)

    zdescs = [pltpu.async_copy(
        zero_v, shared.at[pl.ds(slice0 + r * ZB, ZB)], esem)
        for r in range(CHUNK // NS // ZB)]
    for d in zdescs:
        d.wait()
    plsc.subcore_barrier()

    sdesc = [[], []]

    def fire_scatters(p):
        for r in range(SUB // 128):
            sdesc[p].append(pltpu.async_copy(
                val_v.at[p, r],
                shared.at[plsc.Indices(idx_v.at[p, r], ignored_value=-1)],
                ssem, add=True))

    def drain_scatters(p):
        for d in sdesc[p]:
            d.wait()
        sdesc[p] = []

    base = c * CHUNK

    def unpack_nid(i16):
        return plsc.load_gather(nid_v, [i16])

    def emit(p, g, d16, n16):
        widx = lax.bitwise_or(lax.shift_left(d16, 6),
                              lax.bitwise_and(n16, 63))
        rel = widx - base
        inr = rel.astype(jnp.uint32) < CHUNK
        hi4 = lax.shift_left(lax.shift_right_logical(n16, 6), 4)
        row = g // 8
        col = pl.ds((g % 8) * 16, 16)
        idx_v[p, row, col] = jnp.where(inr, rel, -1)
        val_v[p, row, col] = lax.shift_left(1, hi4)

    for b in range(NSUB_E):
        for d in edesc:
            d.wait()
        if b + 1 < NSUB_E:
            edesc = edge_fetch(b + 1)
        p = b % 2
        drain_scatters(p)

        @plsc.parallel_loop(0, SUB // 16, 1, unroll=4)
        def grp_body(g):
            s16 = src_v[p, pl.ds(g * 16, 16)]
            d16 = dst_v[p, pl.ds(g * 16, 16)]
            emit(p, g, d16, unpack_nid(s16))

        fire_scatters(p)

    p = NSUB_E % 2
    drain_scatters(p)
    self_base = s * (N // NS)
    lane = lax.iota(jnp.int32, 16)

    @plsc.parallel_loop(0, SUB // 16, 1, unroll=4)
    def self_body(g):
        i16 = lane + (self_base + g * 16)
        emit(p, g, i16, unpack_nid(i16))

    fire_scatters(p)

    drain_scatters(0)
    drain_scatters(1)
    plsc.subcore_barrier()
    pltpu.sync_copy(shared.at[pl.ds(slice0, CHUNK // NS)],
                    c_hbm.at[pl.ds(base + slice0, CHUNK // NS)])


def _build_counts(edge_index, node_ids):
    mesh = plsc.VectorSubcoreMesh(core_axis_name="c", subcore_axis_name="s")
    return pl.kernel(
        _hist_body,
        out_type=jax.ShapeDtypeStruct((N * (D // 2),), jnp.int32),
        mesh=mesh,
        compiler_params=pltpu.CompilerParams(needs_layout_passes=False),
        scratch_types=[
            pltpu.VMEM((N,), jnp.int32),
            pltpu.VMEM((2, SUB), jnp.int32),
            pltpu.VMEM((2, SUB), jnp.int32),
            pltpu.VMEM((2, SUB // 128, 128), jnp.int32),
            pltpu.VMEM((2, SUB // 128, 128), jnp.int32),
            pltpu.VMEM((ZB,), jnp.int32),
            pltpu.VMEM_SHARED((CHUNK,), jnp.int32),
            pltpu.SemaphoreType.DMA,
            pltpu.SemaphoreType.DMA,
        ],
    )(edge_index, node_ids)


TRIU = N_NODES * (N_NODES - 1) // 2
GPT = B // (NC * NS)


def _pack_body(dp_hbm, idx_hbm, out_hbm, idx_v, dp_a, dp_b, out_a, out_b,
               dsem, osem):
    c = lax.axis_index("c")
    s = lax.axis_index("s")
    g0 = c * (B // NC) + s * GPT
    pltpu.sync_copy(idx_hbm, idx_v)
    nn2 = N_NODES * N_NODES
    dp_v = [dp_a, dp_b]
    out_v = [out_a, out_b]
    ddesc = [None, None]
    odesc = [None, None]
    ddesc[0] = pltpu.async_copy(dp_hbm.at[pl.ds(g0 * nn2, nn2)], dp_a, dsem)
    for k in range(GPT):
        p = k % 2
        ddesc[p].wait()
        if k + 1 < GPT:
            ddesc[1 - p] = pltpu.async_copy(
                dp_hbm.at[pl.ds((g0 + k + 1) * nn2, nn2)], dp_v[1 - p], dsem)
        if odesc[p] is not None:
            odesc[p].wait()
        dpb = dp_v[p]
        outb = out_v[p]

        @plsc.parallel_loop(0, TRIU // 16, 1, unroll=4)
        def gat(q):
            i16 = idx_v[pl.ds(q * 16, 16)]
            outb[0, pl.ds(q * 16, 16)] = plsc.load_gather(dpb, [i16])

        odesc[p] = pltpu.async_copy(outb,
                                    out_hbm.at[pl.ds(g0 + k, 1), :], osem)
    for d in odesc:
        d.wait()


def _pack_triu(dp_flat, flat_idx):
    mesh = plsc.VectorSubcoreMesh(core_axis_name="c", subcore_axis_name="s")
    return pl.kernel(
        _pack_body,
        out_type=jax.ShapeDtypeStruct((B, TRIU), jnp.float32),
        mesh=mesh,
        compiler_params=pltpu.CompilerParams(needs_layout_passes=False),
        scratch_types=[
            pltpu.VMEM((TRIU,), jnp.int32),
            pltpu.VMEM((N_NODES * N_NODES,), jnp.float32),
            pltpu.VMEM((N_NODES * N_NODES,), jnp.float32),
            pltpu.VMEM((1, TRIU), jnp.float32),
            pltpu.VMEM((1, TRIU), jnp.float32),
            pltpu.SemaphoreType.DMA,
            pltpu.SemaphoreType.DMA,
        ],
    )(dp_flat, flat_idx)


BLK = 2048
G_PER_BLK = BLK // N_NODES
GRID = N // BLK


def _ln(h, g, b):
    m = jnp.mean(h, axis=-1, keepdims=True)
    v = jnp.mean((h - m) ** 2, axis=-1, keepdims=True)
    return (h - m) * lax.rsqrt(v + 1e-5) * g + b


def _mm(a, b):
    return jnp.dot(a, b, preferred_element_type=jnp.float32)


def _dense_body(c_ref, emb_ref, w1_ref, b1_ref, lng_ref, lnb_ref,
                w2_ref, b2_ref, sw1_ref, sb1_ref, sw2_ref, sb2_ref,
                ng_ref, nb_ref, ew1_ref, eb1_ref, elng_ref, elnb_ref,
                ew2_ref, eb2_ref, dp_ref, exit_ref, m_s, sums_s):
    i = pl.program_id(0)

    @pl.when(i == 0)
    def _():
        m_s[:] = jnp.dot(emb_ref[:], w1_ref[:],
                         preferred_element_type=jnp.float32)

    w = c_ref[:]
    lo = lax.bitwise_and(w, 0xFFFF).astype(jnp.float32)
    hi = lax.shift_right_logical(w, 16).astype(jnp.float32)
    cnt = jnp.concatenate(
        [jnp.concatenate([lo[:, :64], hi[:, :64]], axis=1),
         jnp.concatenate([lo[:, 64:], hi[:, 64:]], axis=1)], axis=0)
    h = _mm(cnt, m_s[:]) + b1_ref[:]
    h = _ln(h, lng_ref[:], lnb_ref[:])
    h = jnp.maximum(h, 0.0)
    h = _mm(h, w2_ref[:]) + b2_ref[:]
    h = _mm(h, sw1_ref[:]) + sb1_ref[:]
    h = jnp.maximum(h, 0.0)
    h = _mm(h, sw2_ref[:]) + sb2_ref[:]
    x = _ln(h, ng_ref[:], nb_ref[:])

    scale = 1.0 / math.sqrt(D)
    half = BLK // 2
    hn = N_NODES // 2
    for g in range(G_PER_BLK):
        xg = jnp.concatenate([x[g * hn:(g + 1) * hn, :],
                              x[half + g * hn:half + (g + 1) * hn, :]],
                             axis=0)
        sums_s[i * G_PER_BLK + g, :] = jnp.sum(xg, axis=0)
        dp_ref[g, :, :] = lax.dot_general(
            xg, xg, (((1,), (1,)), ((), ())),
            preferred_element_type=jnp.float32) * scale

    @pl.when(i == GRID - 1)
    def _():
        means = sums_s[:] * (1.0 / N_NODES)
        e = jnp.dot(means, ew1_ref[:],
                    preferred_element_type=jnp.float32) + eb1_ref[:]
        e = _ln(e, elng_ref[:], elnb_ref[:])
        e = jnp.maximum(e, 0.0)
        exit_ref[:] = jnp.dot(e, ew2_ref[:],
                              preferred_element_type=jnp.float32) + eb2_ref[:]


def _dense_stage(counts, emb, w1, b1, lng, lnb, w2, b2,
                 sw1, sb1, sw2, sb2, ng, nb,
                 ew1, eb1, elng, elnb, ew2, eb2):
    wspec = pl.BlockSpec((D, D), lambda i: (0, 0))
    bspec = pl.BlockSpec((1, D), lambda i: (0, 0))
    return pl.pallas_call(
        _dense_body,
        grid=(GRID,),
        in_specs=[
            pl.BlockSpec((BLK // 2, D), lambda i: (i, 0)),
            wspec, wspec, bspec, bspec, bspec,
            wspec, bspec, wspec, bspec, wspec, bspec,
            bspec, bspec,
            wspec, bspec, bspec, bspec,
            pl.BlockSpec((D, 1), lambda i: (0, 0)),
            pl.BlockSpec((1, 1), lambda i: (0, 0)),
        ],
        out_specs=[
            pl.BlockSpec((G_PER_BLK, N_NODES, N_NODES), lambda i: (i, 0, 0)),
            pl.BlockSpec((B, 1), lambda i: (0, 0)),
        ],
        out_shape=[
            jax.ShapeDtypeStruct((B, N_NODES, N_NODES), jnp.float32),
            jax.ShapeDtypeStruct((B, 1), jnp.float32),
        ],
        scratch_shapes=[
            pltpu.VMEM((D, D), jnp.float32),
            pltpu.VMEM((B, D), jnp.float32),
        ],
    )(counts, emb, w1, b1, lng, lnb, w2, b2, sw1, sb1, sw2, sb2, ng, nb,
      ew1, eb1, elng, elnb, ew2, eb2)


def kernel(node_ids, edge_index, ptr, emb, gin_w1, gin_b1, gin_lng, gin_lnb,
           gin_w2, gin_b2, seq_w1, seq_b1, seq_w2, seq_b2, norm_g, norm_b,
           ex_w1, ex_b1, ex_lng, ex_lnb, ex_w2, ex_b2):
    del ptr
    node_ids = node_ids.astype(jnp.int32)

    counts = _build_counts(edge_index.astype(jnp.int32),
                           node_ids).reshape(N // 2, D)

    r2 = lambda v: v.reshape(1, D)
    dp, exit_action = _dense_stage(
        counts, emb, gin_w1, r2(gin_b1), r2(gin_lng), r2(gin_lnb),
        gin_w2, r2(gin_b2), seq_w1, r2(seq_b1), seq_w2, r2(seq_b2),
        r2(norm_g), r2(norm_b),
        ex_w1, r2(ex_b1), r2(ex_lng), r2(ex_lnb), ex_w2,
        ex_b2.reshape(1, 1))

    i0, i1 = np.triu_indices(N_NODES, k=1)
    p0 = (i0 // 2) + (N_NODES // 2) * (i0 % 2)
    p1 = (i1 // 2) + (N_NODES // 2) * (i1 % 2)
    flat_idx = jnp.asarray(p0 * N_NODES + p1, dtype=jnp.int32)
    edge_actions = _pack_triu(dp.reshape(-1), flat_idx)
    return jnp.concatenate([edge_actions, exit_action], axis=-1)


# speedup vs baseline: 1.2529x; 1.0209x over previous
import functools
import math

import jax
import jax.numpy as jnp
import numpy as np
from jax import lax
from jax.experimental import pallas as pl
from jax.experimental.pallas import tpu as pltpu
from jax.experimental.pallas import tpu_sc as plsc

N_NODES = 128
B = 256
N = B * N_NODES
E = 524288
D = 128

NC, NS = 2, 16
SUB = 2048
NSUB_E = E // NS // SUB
NSUB = NSUB_E + 1
CHUNK_ROWS = N // NC
CHUNK = CHUNK_ROWS * (D // 2)
ZB = 8192


def _hist_body(edge_hbm, nid_hbm, c_hbm,
               nid_v, src_v, dst_v, idx_v, val_v, zero_v, shared,
               esem, ssem):
    c = lax.axis_index("c")
    s = lax.axis_index("s")
    tile_base = s * (E // NS)
    slice0 = s * (CHUNK // NS)

    def edge_fetch(b):
        ib = tile_base + b * SUB
        return (pltpu.async_copy(edge_hbm.at[0, pl.ds(ib, SUB)],
                                 src_v.at[b % 2], esem),
                pltpu.async_copy(edge_hbm.at[1, pl.ds(ib, SUB)],
                                 dst_v.at[b % 2], esem))

    edesc = edge_fetch(0)
    pltpu.sync_copy(nid_hbm, nid_v)

    z16 = jnp.zeros((16,), jnp.int32)

    def zb_body(i, _):
        zero_v[pl.ds(i * 16, 16)] = z16
        return 0

    lax.fori_loop(0, ZB // 16, zb_body, 0)

    zdescs = [pltpu.async_copy(
        zero_v, shared.at[pl.ds(slice0 + r * ZB, ZB)], esem)
        for r in range(CHUNK // NS // ZB)]
    for d in zdescs:
        d.wait()
    plsc.subcore_barrier()

    sdesc = [[], []]

    def fire_scatters(p):
        for r in range(SUB // 128):
            sdesc[p].append(pltpu.async_copy(
                val_v.at[p, r],
                shared.at[plsc.Indices(idx_v.at[p, r], ignored_value=-1)],
                ssem, add=True))

    def drain_scatters(p):
        for d in sdesc[p]:
            d.wait()
        sdesc[p] = []

    base = c * CHUNK

    def unpack_nid(i16):
        return plsc.load_gather(nid_v, [i16])

    def emit(p, g, d16, n16):
        widx = lax.bitwise_or(lax.shift_left(d16, 6),
                              lax.bitwise_and(n16, 63))
        rel = widx - base
        inr = rel.astype(jnp.uint32) < CHUNK
        hi4 = lax.shift_left(lax.shift_right_logical(n16, 6), 4)
        row = g // 8
        col = pl.ds((g % 8) * 16, 16)
        idx_v[p, row, col] = jnp.where(inr, rel, -1)
        val_v[p, row, col] = lax.shift_left(1, hi4)

    for b in range(NSUB_E):
        for d in edesc:
            d.wait()
        if b + 1 < NSUB_E:
            edesc = edge_fetch(b + 1)
        p = b % 2
        drain_scatters(p)

        @plsc.parallel_loop(0, SUB // 16, 1, unroll=4)
        def grp_body(g):
            s16 = src_v[p, pl.ds(g * 16, 16)]
            d16 = dst_v[p, pl.ds(g * 16, 16)]
            emit(p, g, d16, unpack_nid(s16))

        fire_scatters(p)

    p = NSUB_E % 2
    drain_scatters(p)
    self_base = s * (N // NS)
    lane = lax.iota(jnp.int32, 16)

    @plsc.parallel_loop(0, SUB // 16, 1, unroll=4)
    def self_body(g):
        i16 = lane + (self_base + g * 16)
        emit(p, g, i16, unpack_nid(i16))

    fire_scatters(p)

    drain_scatters(0)
    drain_scatters(1)
    plsc.subcore_barrier()
    pltpu.sync_copy(shared.at[pl.ds(slice0, CHUNK // NS)],
                    c_hbm.at[pl.ds(base + slice0, CHUNK // NS)])


def _build_counts(edge_index, node_ids):
    mesh = plsc.VectorSubcoreMesh(core_axis_name="c", subcore_axis_name="s")
    return pl.kernel(
        _hist_body,
        out_type=jax.ShapeDtypeStruct((N * (D // 2),), jnp.int32),
        mesh=mesh,
        compiler_params=pltpu.CompilerParams(needs_layout_passes=False),
        scratch_types=[
            pltpu.VMEM((N,), jnp.int32),
            pltpu.VMEM((2, SUB), jnp.int32),
            pltpu.VMEM((2, SUB), jnp.int32),
            pltpu.VMEM((2, SUB // 128, 128), jnp.int32),
            pltpu.VMEM((2, SUB // 128, 128), jnp.int32),
            pltpu.VMEM((ZB,), jnp.int32),
            pltpu.VMEM_SHARED((CHUNK,), jnp.int32),
            pltpu.SemaphoreType.DMA,
            pltpu.SemaphoreType.DMA,
        ],
    )(edge_index, node_ids)


TRIU = N_NODES * (N_NODES - 1) // 2
GPT = B // (NC * NS)


def _pack_body(dp_hbm, idx_hbm, out_hbm, idx_v, dp_a, dp_b, out_a, out_b,
               dsem, osem):
    c = lax.axis_index("c")
    s = lax.axis_index("s")
    g0 = c * (B // NC) + s * GPT
    pltpu.sync_copy(idx_hbm, idx_v)
    nn2 = N_NODES * N_NODES
    dp_v = [dp_a, dp_b]
    out_v = [out_a, out_b]
    ddesc = [None, None]
    odesc = [None, None]
    ddesc[0] = pltpu.async_copy(dp_hbm.at[pl.ds(g0 * nn2, nn2)], dp_a, dsem)
    for k in range(GPT):
        p = k % 2
        ddesc[p].wait()
        if k + 1 < GPT:
            ddesc[1 - p] = pltpu.async_copy(
                dp_hbm.at[pl.ds((g0 + k + 1) * nn2, nn2)], dp_v[1 - p], dsem)
        if odesc[p] is not None:
            odesc[p].wait()
        dpb = dp_v[p]
        outb = out_v[p]

        @plsc.parallel_loop(0, TRIU // 16, 1, unroll=4)
        def gat(q):
            i16 = idx_v[pl.ds(q * 16, 16)]
            outb[0, pl.ds(q * 16, 16)] = plsc.load_gather(dpb, [i16])

        odesc[p] = pltpu.async_copy(outb,
                                    out_hbm.at[pl.ds(g0 + k, 1), :], osem)
    for d in odesc:
        d.wait()


def _pack_triu(dp_flat, flat_idx):
    mesh = plsc.VectorSubcoreMesh(core_axis_name="c", subcore_axis_name="s")
    return pl.kernel(
        _pack_body,
        out_type=jax.ShapeDtypeStruct((B, TRIU), jnp.float32),
        mesh=mesh,
        compiler_params=pltpu.CompilerParams(needs_layout_passes=False),
        scratch_types=[
            pltpu.VMEM((TRIU,), jnp.int32),
            pltpu.VMEM((N_NODES * N_NODES,), jnp.float32),
            pltpu.VMEM((N_NODES * N_NODES,), jnp.float32),
            pltpu.VMEM((1, TRIU), jnp.float32),
            pltpu.VMEM((1, TRIU), jnp.float32),
            pltpu.SemaphoreType.DMA,
            pltpu.SemaphoreType.DMA,
        ],
    )(dp_flat, flat_idx)


BLK = 4096
G_PER_BLK = BLK // N_NODES
GRID = N // BLK


def _ln(h, g, b):
    m = jnp.mean(h, axis=-1, keepdims=True)
    v = jnp.mean((h - m) ** 2, axis=-1, keepdims=True)
    return (h - m) * lax.rsqrt(v + 1e-5) * g + b


def _mm(a, b):
    return jnp.dot(a, b, preferred_element_type=jnp.float32)


def _dense_body(c_ref, emb_ref, w1_ref, b1_ref, lng_ref, lnb_ref,
                w2_ref, b2_ref, sw1_ref, sb1_ref, sw2_ref, sb2_ref,
                ng_ref, nb_ref, ew1_ref, eb1_ref, elng_ref, elnb_ref,
                ew2_ref, eb2_ref, dp_ref, exit_ref, m_s, sums_s):
    i = pl.program_id(0)

    @pl.when(i == 0)
    def _():
        m_s[:] = jnp.dot(emb_ref[:], w1_ref[:],
                         preferred_element_type=jnp.float32)

    w = c_ref[:]
    lo = lax.bitwise_and(w, 0xFFFF).astype(jnp.float32)
    hi = lax.shift_right_logical(w, 16).astype(jnp.float32)
    cnt = jnp.concatenate(
        [jnp.concatenate([lo[:, :64], hi[:, :64]], axis=1),
         jnp.concatenate([lo[:, 64:], hi[:, 64:]], axis=1)], axis=0)
    h = _mm(cnt, m_s[:]) + b1_ref[:]
    h = _ln(h, lng_ref[:], lnb_ref[:])
    h = jnp.maximum(h, 0.0)
    h = _mm(h, w2_ref[:]) + b2_ref[:]
    h = _mm(h, sw1_ref[:]) + sb1_ref[:]
    h = jnp.maximum(h, 0.0)
    h = _mm(h, sw2_ref[:]) + sb2_ref[:]
    x = _ln(h, ng_ref[:], nb_ref[:])

    scale = 1.0 / math.sqrt(D)
    half = BLK // 2
    hn = N_NODES // 2
    for g in range(G_PER_BLK):
        xg = jnp.concatenate([x[g * hn:(g + 1) * hn, :],
                              x[half + g * hn:half + (g + 1) * hn, :]],
                             axis=0)
        sums_s[i * G_PER_BLK + g, :] = jnp.sum(xg, axis=0)
        dp_ref[g, :, :] = lax.dot_general(
            xg, xg, (((1,), (1,)), ((), ())),
            preferred_element_type=jnp.float32) * scale

    @pl.when(i == GRID - 1)
    def _():
        means = sums_s[:] * (1.0 / N_NODES)
        e = jnp.dot(means, ew1_ref[:],
                    preferred_element_type=jnp.float32) + eb1_ref[:]
        e = _ln(e, elng_ref[:], elnb_ref[:])
        e = jnp.maximum(e, 0.0)
        exit_ref[:] = jnp.dot(e, ew2_ref[:],
                              preferred_element_type=jnp.float32) + eb2_ref[:]


def _dense_stage(counts, emb, w1, b1, lng, lnb, w2, b2,
                 sw1, sb1, sw2, sb2, ng, nb,
                 ew1, eb1, elng, elnb, ew2, eb2):
    wspec = pl.BlockSpec((D, D), lambda i: (0, 0))
    bspec = pl.BlockSpec((1, D), lambda i: (0, 0))
    return pl.pallas_call(
        _dense_body,
        grid=(GRID,),
        in_specs=[
            pl.BlockSpec((BLK // 2, D), lambda i: (i, 0)),
            wspec, wspec, bspec, bspec, bspec,
            wspec, bspec, wspec, bspec, wspec, bspec,
            bspec, bspec,
            wspec, bspec, bspec, bspec,
            pl.BlockSpec((D, 1), lambda i: (0, 0)),
            pl.BlockSpec((1, 1), lambda i: (0, 0)),
        ],
        out_specs=[
            pl.BlockSpec((G_PER_BLK, N_NODES, N_NODES), lambda i: (i, 0, 0)),
            pl.BlockSpec((B, 1), lambda i: (0, 0)),
        ],
        out_shape=[
            jax.ShapeDtypeStruct((B, N_NODES, N_NODES), jnp.float32),
            jax.ShapeDtypeStruct((B, 1), jnp.float32),
        ],
        scratch_shapes=[
            pltpu.VMEM((D, D), jnp.float32),
            pltpu.VMEM((B, D), jnp.float32),
        ],
    )(counts, emb, w1, b1, lng, lnb, w2, b2, sw1, sb1, sw2, sb2, ng, nb,
      ew1, eb1, elng, elnb, ew2, eb2)


def kernel(node_ids, edge_index, ptr, emb, gin_w1, gin_b1, gin_lng, gin_lnb,
           gin_w2, gin_b2, seq_w1, seq_b1, seq_w2, seq_b2, norm_g, norm_b,
           ex_w1, ex_b1, ex_lng, ex_lnb, ex_w2, ex_b2):
    del ptr
    node_ids = node_ids.astype(jnp.int32)

    counts = _build_counts(edge_index.astype(jnp.int32),
                           node_ids).reshape(N // 2, D)

    r2 = lambda v: v.reshape(1, D)
    dp, exit_action = _dense_stage(
        counts, emb, gin_w1, r2(gin_b1), r2(gin_lng), r2(gin_lnb),
        gin_w2, r2(gin_b2), seq_w1, r2(seq_b1), seq_w2, r2(seq_b2),
        r2(norm_g), r2(norm_b),
        ex_w1, r2(ex_b1), r2(ex_lng), r2(ex_lnb), ex_w2,
        ex_b2.reshape(1, 1))

    i0, i1 = np.triu_indices(N_NODES, k=1)
    p0 = (i0 // 2) + (N_NODES // 2) * (i0 % 2)
    p1 = (i1 // 2) + (N_NODES // 2) * (i1 % 2)
    flat_idx = jnp.asarray(p0 * N_NODES + p1, dtype=jnp.int32)
    edge_actions = _pack_triu(dp.reshape(-1), flat_idx)
    return jnp.concatenate([edge_actions, exit_action], axis=-1)
